# baseline (speedup 1.0000x reference)
import functools

import jax
import jax.numpy as jnp
from jax.experimental import pallas as pl
from jax.experimental.pallas import tpu as pltpu

N_NODES = 10000
N_EDGES = 320000
H = 128
R = 6
K = 8
NH = 4
HD = H // NH
SCALE = float(jnp.sqrt(jnp.float32(HD)))
CHUNK = 512
N_CHUNKS = N_EDGES // CHUNK
NBLK = 2000


def _proj_body(h_ref, wq_ref, wk_ref, wv_ref, qt_ref, kt_ref, vt_ref):
    hmat = h_ref[...]
    dn = (((1,), (1,)), ((), ()))
    qt_ref[0] = jax.lax.dot_general(hmat, wq_ref[0], dn,
                                    preferred_element_type=jnp.float32)
    kt_ref[0] = jax.lax.dot_general(hmat, wk_ref[0], dn,
                                    preferred_element_type=jnp.float32)
    vt_ref[0] = jax.lax.dot_general(hmat, wv_ref[0], dn,
                                    preferred_element_type=jnp.float32)


def _edge_body(ei_ref, et_ref, qt_ref, kt_ref, vt_ref, num_ref, den_ref):
    r = pl.program_id(0)
    c = pl.program_id(1)

    @pl.when(c == 0)
    def _():
        num_ref[...] = jnp.zeros_like(num_ref)
        den_ref[...] = jnp.zeros_like(den_ref)

    ones32 = jnp.ones((1, 32), jnp.float32)
    inv_scale = 1.0 / SCALE

    def body(e, _):
        t = et_ref[e]

        @pl.when(t == r)
        def _():
            s = ei_ref[0, e]
            d = ei_ref[1, e]
            q = qt_ref[0, pl.ds(d, 1), :]
            k = kt_ref[0, pl.ds(s, 1), :]
            v = vt_ref[0, pl.ds(s, 1), :]
            qk = q * k
            s0 = jnp.sum(qk[:, 0:32]) * inv_scale
            s1 = jnp.sum(qk[:, 32:64]) * inv_scale
            s2 = jnp.sum(qk[:, 64:96]) * inv_scale
            s3 = jnp.sum(qk[:, 96:128]) * inv_scale
            w = jnp.exp(jnp.concatenate(
                [s0 * ones32, s1 * ones32, s2 * ones32, s3 * ones32], axis=1))
            num_ref[0, pl.ds(d, 1), :] += w * v
            den_ref[0, pl.ds(d, 1), :] += w

        return 0

    jax.lax.fori_loop(0, CHUNK, body, 0)


def _epi_body(num_ref, den_ref, h_ref, w1_ref, b1_ref, w2_ref, b2_ref,
              sym_ref, wsw_ref, wsb_ref, wc_ref, bc_ref, wg_ref, bg_ref,
              gamma_ref, beta_ref, out_ref):
    dn = (((1,), (1,)), ((), ()))
    num = num_ref[...]
    den = den_ref[...]
    agg = jnp.where(den > 0.0, num / jnp.where(den > 0.0, den, 1.0), 0.0)
    hmat = h_ref[...]

    hidden = jnp.maximum(
        jax.lax.dot_general(hmat, w1_ref[...], dn,
                            preferred_element_type=jnp.float32) + b1_ref[...],
        0.0)
    logits = jax.lax.dot_general(hidden, w2_ref[...], dn,
                                 preferred_element_type=jnp.float32) + b2_ref[...]
    m = jnp.max(logits, axis=-1, keepdims=True)
    ex = jnp.exp(logits - m)
    gate_vals = ex / jnp.sum(ex, axis=-1, keepdims=True)

    sym = sym_ref[...]
    sm = jnp.max(sym, axis=1, keepdims=True)
    sex = jnp.exp(sym - sm)
    sym_w = sex / jnp.sum(sex, axis=1, keepdims=True)

    channels = []
    for kk in range(K):
        s_k = agg[0] * sym_w[kk, 0]
        for rr in range(1, R):
            s_k = s_k + agg[rr] * sym_w[kk, rr]
        s_k = jnp.maximum(
            jax.lax.dot_general(s_k, wsw_ref[kk], dn,
                                preferred_element_type=jnp.float32)
            + wsb_ref[kk:kk + 1, :], 0.0)
        channels.append(s_k * gate_vals[:, kk:kk + 1])
    cat = jnp.concatenate(channels, axis=-1)
    h_new = jnp.maximum(
        jax.lax.dot_general(cat, wc_ref[...], dn,
                            preferred_element_type=jnp.float32) + bc_ref[...],
        0.0)
    cat2 = jnp.concatenate([hmat, h_new], axis=-1)
    gate = jax.nn.sigmoid(
        jax.lax.dot_general(cat2, wg_ref[...], dn,
                            preferred_element_type=jnp.float32) + bg_ref[...])
    x = gate * h_new + (1.0 - gate) * hmat
    mu = jnp.mean(x, axis=-1, keepdims=True)
    var = jnp.mean((x - mu) ** 2, axis=-1, keepdims=True)
    out_ref[...] = (x - mu) / jnp.sqrt(var + 1e-5) * gamma_ref[...] + beta_ref[...]


@jax.jit
def kernel(h, edge_index, edge_type, W_q, W_k, W_v, mlp_W1, mlp_b1, mlp_W2,
           mlp_b2, sym_edge_logits, Ws_W, Ws_b, Wc, bc, Wg, bg, gamma, beta):
    f32 = jnp.float32

    qt, kt, vt = pl.pallas_call(
        _proj_body,
        grid=(R,),
        in_specs=[
            pl.BlockSpec((N_NODES, H), lambda r: (0, 0)),
            pl.BlockSpec((1, H, H), lambda r: (r, 0, 0)),
            pl.BlockSpec((1, H, H), lambda r: (r, 0, 0)),
            pl.BlockSpec((1, H, H), lambda r: (r, 0, 0)),
        ],
        out_specs=[
            pl.BlockSpec((1, N_NODES, H), lambda r: (r, 0, 0)),
            pl.BlockSpec((1, N_NODES, H), lambda r: (r, 0, 0)),
            pl.BlockSpec((1, N_NODES, H), lambda r: (r, 0, 0)),
        ],
        out_shape=[
            jax.ShapeDtypeStruct((R, N_NODES, H), f32),
            jax.ShapeDtypeStruct((R, N_NODES, H), f32),
            jax.ShapeDtypeStruct((R, N_NODES, H), f32),
        ],
    )(h, W_q, W_k, W_v)

    num, den = pl.pallas_call(
        _edge_body,
        grid=(R, N_CHUNKS),
        in_specs=[
            pl.BlockSpec((2, CHUNK), lambda r, c: (0, c),
                         memory_space=pltpu.SMEM),
            pl.BlockSpec((CHUNK,), lambda r, c: (c,),
                         memory_space=pltpu.SMEM),
            pl.BlockSpec((1, N_NODES, H), lambda r, c: (r, 0, 0)),
            pl.BlockSpec((1, N_NODES, H), lambda r, c: (r, 0, 0)),
            pl.BlockSpec((1, N_NODES, H), lambda r, c: (r, 0, 0)),
        ],
        out_specs=[
            pl.BlockSpec((1, N_NODES, H), lambda r, c: (r, 0, 0)),
            pl.BlockSpec((1, N_NODES, H), lambda r, c: (r, 0, 0)),
        ],
        out_shape=[
            jax.ShapeDtypeStruct((R, N_NODES, H), f32),
            jax.ShapeDtypeStruct((R, N_NODES, H), f32),
        ],
    )(edge_index, edge_type, qt, kt, vt)

    full = lambda shape: pl.BlockSpec(shape, lambda i: tuple(0 for _ in shape))
    out = pl.pallas_call(
        _epi_body,
        grid=(N_NODES // NBLK,),
        in_specs=[
            pl.BlockSpec((R, NBLK, H), lambda i: (0, i, 0)),
            pl.BlockSpec((R, NBLK, H), lambda i: (0, i, 0)),
            pl.BlockSpec((NBLK, H), lambda i: (i, 0)),
            full((H, H)),
            full((1, H)),
            full((K, H)),
            full((1, K)),
            full((K, R)),
            full((K, H, H)),
            full((K, H)),
            full((H, K * H)),
            full((1, H)),
            full((H, 2 * H)),
            full((1, H)),
            full((1, H)),
            full((1, H)),
        ],
        out_specs=pl.BlockSpec((NBLK, H), lambda i: (i, 0)),
        out_shape=jax.ShapeDtypeStruct((N_NODES, H), f32),
    )(num, den, h, mlp_W1, mlp_b1.reshape(1, H), mlp_W2, mlp_b2.reshape(1, K),
      sym_edge_logits, Ws_W, Ws_b, Wc, bc.reshape(1, H), Wg, bg.reshape(1, H),
      gamma.reshape(1, H), beta.reshape(1, H))
    return out
